# Initial kernel scaffold; baseline (speedup 1.0000x reference)
import functools

import jax
import jax.numpy as jnp
from jax import lax
from jax.experimental import pallas as pl
from jax.experimental.pallas import tpu as pltpu
from jax.experimental.pallas import tpu_sc as plsc

_N = 10000
_NP = 10240
_E = 320000
_EP = 327680
_ROWS = _EP // 128
_NC = 2
_NS = 16
_NW = _NC * _NS
_NR = _ROWS // _NW
_NSL = _NP // _NS
_B = 4
_LAG = 2

_mesh = plsc.VectorSubcoreMesh(core_axis_name="c", subcore_axis_name="s")


def _make_deg():
    @functools.partial(
        pl.kernel,
        out_type=jax.ShapeDtypeStruct((_NC, 2, _NP), jnp.float32),
        mesh=_mesh,
        scratch_types=[
            pltpu.VMEM((2 * _NR, 128), jnp.int32),
            pltpu.VMEM((128,), jnp.float32),
            pltpu.VMEM_SHARED((_NP,), jnp.float32),
            pltpu.VMEM_SHARED((_NP,), jnp.float32),
            pltpu.SemaphoreType.DMA((8,)),
        ],
    )
    def deg(srcp, dstp, zdeg, out, idx_v, ones_v, dsrc, ddst, sems):
        c = lax.axis_index("c")
        s = lax.axis_index("s")
        wid = c * _NS + s
        row0 = wid * _NR
        nd0 = s * _NSL
        pltpu.sync_copy(zdeg.at[pl.ds(nd0, _NSL)], dsrc.at[pl.ds(nd0, _NSL)])
        pltpu.sync_copy(zdeg.at[pl.ds(nd0, _NSL)], ddst.at[pl.ds(nd0, _NSL)])
        pltpu.sync_copy(srcp.at[pl.ds(row0, _NR)], idx_v.at[pl.ds(0, _NR)])
        pltpu.sync_copy(dstp.at[pl.ds(row0, _NR)], idx_v.at[pl.ds(_NR, _NR)])
        for i in range(8):
            ones_v[pl.ds(i * 16, 16)] = jnp.full((16,), 1.0, jnp.float32)
        plsc.subcore_barrier()
        descs = [None] * 8
        for j in range(2 * _NR):
            k = j % 8
            if descs[k] is not None:
                descs[k].wait()
            tgt = dsrc if j < _NR else ddst
            descs[k] = pltpu.async_copy(ones_v, tgt.at[idx_v.at[j]],
                                        sems.at[k], add=True)
        for k in range(8):
            descs[k].wait()
        plsc.subcore_barrier()
        pltpu.sync_copy(dsrc.at[pl.ds(nd0, _NSL)], out.at[c, 0, pl.ds(nd0, _NSL)])
        pltpu.sync_copy(ddst.at[pl.ds(nd0, _NSL)], out.at[c, 1, pl.ds(nd0, _NSL)])

    return deg


def _make_prop(w):
    @functools.partial(
        pl.kernel,
        out_type=jax.ShapeDtypeStruct((_NC, _NP, w), jnp.float32),
        mesh=_mesh,
        scratch_types=[
            pltpu.VMEM((2 * _NR, 128), jnp.int32),
            pltpu.VMEM((_B, 128, w), jnp.float32),
            pltpu.VMEM_SHARED((_NP, w), jnp.float32),
            pltpu.VMEM_SHARED((_NP, w), jnp.float32),
            pltpu.SemaphoreType.DMA((2 * _B,)),
        ],
    )
    def prop(table_hbm, srcp, dstp, ztab, out, idx_v, rb, table, acc, sems):
        c = lax.axis_index("c")
        s = lax.axis_index("s")
        wid = c * _NS + s
        row0 = wid * _NR
        nd0 = s * _NSL
        pltpu.sync_copy(table_hbm.at[pl.ds(nd0, _NSL)], table.at[pl.ds(nd0, _NSL)])
        pltpu.sync_copy(ztab.at[pl.ds(nd0, _NSL)], acc.at[pl.ds(nd0, _NSL)])
        pltpu.sync_copy(srcp.at[pl.ds(row0, _NR)], idx_v.at[pl.ds(0, _NR)])
        pltpu.sync_copy(dstp.at[pl.ds(row0, _NR)], idx_v.at[pl.ds(_NR, _NR)])
        plsc.subcore_barrier()
        gd = [None] * _B
        sd = [None] * _B
        for t in range(_NR + _LAG):
            if t >= _LAG:
                j = t - _LAG
                slot = j % _B
                gd[slot].wait()
                sd[slot] = pltpu.async_copy(rb.at[slot], acc.at[idx_v.at[_NR + j]],
                                            sems.at[_B + slot], add=True)
            if t < _NR:
                slot = t % _B
                if sd[slot] is not None:
                    sd[slot].wait()
                    sd[slot] = None
                gd[slot] = pltpu.async_copy(table.at[idx_v.at[t]], rb.at[slot],
                                            sems.at[slot])
        for slot in range(_B):
            if sd[slot] is not None:
                sd[slot].wait()
        plsc.subcore_barrier()
        pltpu.sync_copy(acc.at[pl.ds(nd0, _NSL)], out.at[c, pl.ds(nd0, _NSL)])

    return prop


_DEG = _make_deg()
_P16 = _make_prop(16)
_P48 = _make_prop(48)


def _tca_body(feat_ref, w1_ref, degs_ref, u0_ref, ns_ref, nd_ref):
    deg_out = degs_ref[0, 0] + degs_ref[1, 0]
    deg_in = degs_ref[0, 1] + degs_ref[1, 1]
    ns = jnp.where(deg_out > 0, lax.rsqrt(jnp.maximum(deg_out, 1e-12)), 0.0)
    nd = jnp.where(deg_in > 0, lax.rsqrt(jnp.maximum(deg_in, 1e-12)), 0.0)
    y = jnp.dot(feat_ref[...], w1_ref[...], preferred_element_type=jnp.float32)
    u0_ref[...] = y * ns
    ns_ref[...] = ns
    nd_ref[...] = nd


def _tct2_body(parts_ref, ns_ref, nd_ref, b_ref, h_ref, u_ref):
    agg = parts_ref[0] + parts_ref[1]
    h = jnp.maximum(agg * nd_ref[...] + b_ref[...], 0.0)
    h_ref[...] = h
    u_ref[...] = h * ns_ref[...]


def _tct3_body(parts_ref, w_ref, ns_ref, nd_ref, b_ref, h_ref, u_ref):
    agg = jnp.dot(parts_ref[0] + parts_ref[1], w_ref[...],
                  preferred_element_type=jnp.float32)
    h = jnp.maximum(agg * nd_ref[...] + b_ref[...], 0.0)
    h_ref[...] = h
    u_ref[...] = h * ns_ref[...]


def _tct4_body(parts_ref, w_ref, nd_ref, b_ref, h1_ref, h2_ref, t48_ref):
    agg = jnp.dot(parts_ref[0] + parts_ref[1], w_ref[...],
                  preferred_element_type=jnp.float32)
    h3 = jnp.maximum(agg * nd_ref[...] + b_ref[...], 0.0)
    t48_ref[...] = jnp.concatenate([h1_ref[...], h2_ref[...], h3], axis=1)


def _tcf_body(parts_ref, wm_ref, bm_ref, out_ref):
    out_ref[...] = jnp.dot(parts_ref[0] + parts_ref[1], wm_ref[...],
                           preferred_element_type=jnp.float32) + bm_ref[...]


def _sds(shape):
    return jax.ShapeDtypeStruct(shape, jnp.float32)


def kernel(feat, edge_index, W1, b1, W2, b2, W3, b3, Wm, bm):
    f32 = jnp.float32
    pad_ids = _N + (jnp.arange(_EP - _E, dtype=jnp.int32) % (_NP - _N))
    srcp = jnp.concatenate([edge_index[0], pad_ids]).reshape(_ROWS, 128)
    dstp = jnp.concatenate([edge_index[1], pad_ids]).reshape(_ROWS, 128)
    featp = jnp.concatenate(
        [feat.astype(f32), jnp.zeros((_NP - _N, feat.shape[1]), f32)], axis=0)
    zdeg = jnp.zeros((_NP,), f32)
    z16 = jnp.zeros((_NP, 16), f32)
    z48 = jnp.zeros((_NP, 48), f32)

    degs = _DEG(srcp, dstp, zdeg).reshape(_NC, 2, _NP, 1)
    u0, ns, nd = pl.pallas_call(
        _tca_body,
        out_shape=(_sds((_NP, 16)), _sds((_NP, 1)), _sds((_NP, 1))),
    )(featp, W1, degs)

    p1 = _P16(u0, srcp, dstp, z16)
    h1, u1 = pl.pallas_call(
        _tct2_body, out_shape=(_sds((_NP, 16)), _sds((_NP, 16))),
    )(p1, ns, nd, b1.reshape(1, 16))

    p2 = _P16(u1, srcp, dstp, z16)
    h2, u2 = pl.pallas_call(
        _tct3_body, out_shape=(_sds((_NP, 16)), _sds((_NP, 16))),
    )(p2, W2, ns, nd, b2.reshape(1, 16))

    p3 = _P16(u2, srcp, dstp, z16)
    t48 = pl.pallas_call(
        _tct4_body, out_shape=_sds((_NP, 48)),
    )(p3, W3, nd, b3.reshape(1, 16), h1, h2)

    p4 = _P48(t48, srcp, dstp, z48)
    outp = pl.pallas_call(
        _tcf_body, out_shape=_sds((_NP, 128)),
    )(p4, Wm, bm.reshape(1, 128))
    return outp[:_N]


# speedup vs baseline: 19.8326x; 19.8326x over previous
import functools

import jax
import jax.numpy as jnp
from jax import lax
from jax.experimental import pallas as pl
from jax.experimental.pallas import tpu as pltpu
from jax.experimental.pallas import tpu_sc as plsc

_N = 10000
_NP = 10240
_E = 320000
_EP = 327680
_ROWS = _EP // 128
_NC = 2
_NS = 16
_NW = _NC * _NS
_NR = _ROWS // _NW
_NSL = _NP // _NS
_B = 4
_LAG = 2

_mesh = plsc.VectorSubcoreMesh(core_axis_name="c", subcore_axis_name="s")
_sc_params = pltpu.CompilerParams(use_tc_tiling_on_sc=False)


def _make_deg():
    @functools.partial(
        pl.kernel,
        out_type=jax.ShapeDtypeStruct((_NC * 2 * _NP,), jnp.float32),
        mesh=_mesh,
        scratch_types=[
            pltpu.VMEM((2 * _NR, 128), jnp.int32),
            pltpu.VMEM((128,), jnp.float32),
            pltpu.VMEM_SHARED((_NP,), jnp.float32),
            pltpu.VMEM_SHARED((_NP,), jnp.float32),
            pltpu.SemaphoreType.DMA((8,)),
        ],
        compiler_params=_sc_params,
    )
    def deg(srcp, dstp, zdeg, out, idx_v, ones_v, dsrc, ddst, sems):
        c = lax.axis_index("c")
        s = lax.axis_index("s")
        wid = c * _NS + s
        row0 = wid * _NR
        nd0 = s * _NSL
        pltpu.sync_copy(zdeg.at[pl.ds(nd0, _NSL)], dsrc.at[pl.ds(nd0, _NSL)])
        pltpu.sync_copy(zdeg.at[pl.ds(nd0, _NSL)], ddst.at[pl.ds(nd0, _NSL)])
        pltpu.sync_copy(srcp.at[pl.ds(row0, _NR)], idx_v.at[pl.ds(0, _NR)])
        pltpu.sync_copy(dstp.at[pl.ds(row0, _NR)], idx_v.at[pl.ds(_NR, _NR)])
        for i in range(8):
            ones_v[pl.ds(i * 16, 16)] = jnp.full((16,), 1.0, jnp.float32)
        plsc.subcore_barrier()
        descs = [None] * 8
        for j in range(2 * _NR):
            k = j % 8
            if descs[k] is not None:
                descs[k].wait()
            tgt = dsrc if j < _NR else ddst
            descs[k] = pltpu.async_copy(ones_v, tgt.at[idx_v.at[j]],
                                        sems.at[k], add=True)
        for k in range(8):
            descs[k].wait()
        plsc.subcore_barrier()
        pltpu.sync_copy(dsrc.at[pl.ds(nd0, _NSL)],
                        out.at[pl.ds(c * 2 * _NP + nd0, _NSL)])
        pltpu.sync_copy(ddst.at[pl.ds(nd0, _NSL)],
                        out.at[pl.ds(c * 2 * _NP + _NP + nd0, _NSL)])

    return deg


def _make_prop(w):
    nflat = _NSL * w

    @functools.partial(
        pl.kernel,
        out_type=jax.ShapeDtypeStruct((_NC * _NP * w,), jnp.float32),
        mesh=_mesh,
        scratch_types=[
            pltpu.VMEM((2 * _NR, 128), jnp.int32),
            pltpu.VMEM((_B, 128, w), jnp.float32),
            pltpu.VMEM((nflat,), jnp.float32),
            pltpu.VMEM((_NSL, w), jnp.float32),
            pltpu.VMEM_SHARED((_NP, w), jnp.float32),
            pltpu.VMEM_SHARED((_NP, w), jnp.float32),
            pltpu.SemaphoreType.DMA((2 * _B,)),
        ],
        compiler_params=_sc_params,
    )
    def prop(table_hbm, srcp, dstp, out, idx_v, rb, flat, rows, table, acc, sems):
        c = lax.axis_index("c")
        s = lax.axis_index("s")
        wid = c * _NS + s
        row0 = wid * _NR
        nd0 = s * _NSL
        pltpu.sync_copy(table_hbm.at[pl.ds(nd0 * w, nflat)], flat)
        pltpu.sync_copy(srcp.at[pl.ds(row0, _NR)], idx_v.at[pl.ds(0, _NR)])
        pltpu.sync_copy(dstp.at[pl.ds(row0, _NR)], idx_v.at[pl.ds(_NR, _NR)])
        for k in range(_NSL):
            rows[k, :] = flat[pl.ds(k * w, w)]
        pltpu.sync_copy(rows, table.at[pl.ds(nd0, _NSL)])
        zz = jnp.zeros((w,), jnp.float32)
        for k in range(_NSL):
            rows[k, :] = zz
        pltpu.sync_copy(rows, acc.at[pl.ds(nd0, _NSL)])
        plsc.subcore_barrier()
        gd = [None] * _B
        sd = [None] * _B
        for t in range(_NR + _LAG):
            if t >= _LAG:
                j = t - _LAG
                slot = j % _B
                gd[slot].wait()
                sd[slot] = pltpu.async_copy(rb.at[slot], acc.at[idx_v.at[_NR + j]],
                                            sems.at[_B + slot], add=True)
            if t < _NR:
                slot = t % _B
                if sd[slot] is not None:
                    sd[slot].wait()
                    sd[slot] = None
                gd[slot] = pltpu.async_copy(table.at[idx_v.at[t]], rb.at[slot],
                                            sems.at[slot])
        for slot in range(_B):
            if sd[slot] is not None:
                sd[slot].wait()
        plsc.subcore_barrier()
        pltpu.sync_copy(acc.at[pl.ds(nd0, _NSL)], rows)
        for k in range(_NSL):
            flat[pl.ds(k * w, w)] = rows[k, :]
        pltpu.sync_copy(flat, out.at[pl.ds((c * _NP + nd0) * w, nflat)])

    return prop


_DEG = _make_deg()
_P16 = _make_prop(16)


def _tca_body(feat_ref, w1_ref, degs_ref, u0_ref, ns_ref, nd_ref):
    deg_out = degs_ref[0, 0] + degs_ref[1, 0]
    deg_in = degs_ref[0, 1] + degs_ref[1, 1]
    ns = jnp.where(deg_out > 0, lax.rsqrt(jnp.maximum(deg_out, 1e-12)), 0.0)
    nd = jnp.where(deg_in > 0, lax.rsqrt(jnp.maximum(deg_in, 1e-12)), 0.0)
    y = jnp.dot(feat_ref[...], w1_ref[...], preferred_element_type=jnp.float32)
    u0_ref[...] = y * ns
    ns_ref[...] = ns
    nd_ref[...] = nd


def _tct2_body(parts_ref, ns_ref, nd_ref, b_ref, h_ref, u_ref):
    agg = parts_ref[0] + parts_ref[1]
    h = jnp.maximum(agg * nd_ref[...] + b_ref[...], 0.0)
    h_ref[...] = h
    u_ref[...] = h * ns_ref[...]


def _tct3_body(parts_ref, w_ref, ns_ref, nd_ref, b_ref, h_ref, u_ref):
    agg = jnp.dot(parts_ref[0] + parts_ref[1], w_ref[...],
                  preferred_element_type=jnp.float32)
    h = jnp.maximum(agg * nd_ref[...] + b_ref[...], 0.0)
    h_ref[...] = h
    u_ref[...] = h * ns_ref[...]


def _tct4_body(parts_ref, w_ref, nd_ref, b_ref, h_ref):
    agg = jnp.dot(parts_ref[0] + parts_ref[1], w_ref[...],
                  preferred_element_type=jnp.float32)
    h_ref[...] = jnp.maximum(agg * nd_ref[...] + b_ref[...], 0.0)


def _tcf_body(pa_ref, pb_ref, pc_ref, wm_ref, bm_ref, out_ref):
    agg = jnp.concatenate(
        [pa_ref[0] + pa_ref[1], pb_ref[0] + pb_ref[1], pc_ref[0] + pc_ref[1]],
        axis=1)
    out_ref[...] = jnp.dot(agg, wm_ref[...],
                           preferred_element_type=jnp.float32) + bm_ref[...]


def _sds(shape):
    return jax.ShapeDtypeStruct(shape, jnp.float32)


def _prop16(table, srcp, dstp):
    return _P16(table.reshape(-1), srcp, dstp).reshape(_NC, _NP, 16)


def kernel(feat, edge_index, W1, b1, W2, b2, W3, b3, Wm, bm):
    f32 = jnp.float32
    pad_ids = _N + (jnp.arange(_EP - _E, dtype=jnp.int32) % (_NP - _N))
    srcp = jnp.concatenate([edge_index[0], pad_ids]).reshape(_ROWS, 128)
    dstp = jnp.concatenate([edge_index[1], pad_ids]).reshape(_ROWS, 128)
    featp = jnp.concatenate(
        [feat.astype(f32), jnp.zeros((_NP - _N, feat.shape[1]), f32)], axis=0)
    zdeg = jnp.zeros((_NP,), f32)

    degs = _DEG(srcp, dstp, zdeg).reshape(_NC, 2, _NP, 1)
    u0, ns, nd = pl.pallas_call(
        _tca_body,
        out_shape=(_sds((_NP, 16)), _sds((_NP, 1)), _sds((_NP, 1))),
    )(featp, W1, degs)

    p1 = _prop16(u0, srcp, dstp)
    h1, u1 = pl.pallas_call(
        _tct2_body, out_shape=(_sds((_NP, 16)), _sds((_NP, 16))),
    )(p1, ns, nd, b1.reshape(1, 16))

    p2 = _prop16(u1, srcp, dstp)
    h2, u2 = pl.pallas_call(
        _tct3_body, out_shape=(_sds((_NP, 16)), _sds((_NP, 16))),
    )(p2, W2, ns, nd, b2.reshape(1, 16))

    p3 = _prop16(u2, srcp, dstp)
    h3 = pl.pallas_call(
        _tct4_body, out_shape=_sds((_NP, 16)),
    )(p3, W3, nd, b3.reshape(1, 16))

    p4a = _prop16(h1, srcp, dstp)
    p4b = _prop16(h2, srcp, dstp)
    p4c = _prop16(h3, srcp, dstp)
    outp = pl.pallas_call(
        _tcf_body, out_shape=_sds((_NP, 128)),
    )(p4a, p4b, p4c, Wm, bm.reshape(1, 128))
    return outp[:_N]


# speedup vs baseline: 26.0006x; 1.3110x over previous
import functools

import jax
import jax.numpy as jnp
from jax import lax
from jax.experimental import pallas as pl
from jax.experimental.pallas import tpu as pltpu
from jax.experimental.pallas import tpu_sc as plsc

_N = 10000
_NP = 10240
_PK = _NP * 16 // 128
_E = 320000
_EP = 327680
_ROWS = _EP // 128
_NC = 2
_NS = 16
_NW = _NC * _NS
_NR = _ROWS // _NW
_NSL = _NP // _NS
_PKT = _PK // _NS
_B = 4
_LAG = 2

_mesh = plsc.VectorSubcoreMesh(core_axis_name="c", subcore_axis_name="s")
_sc_params = pltpu.CompilerParams(use_tc_tiling_on_sc=False)


def _make_deg():
    @functools.partial(
        pl.kernel,
        out_type=jax.ShapeDtypeStruct((_NC, 2, _PK, 128), jnp.float32),
        mesh=_mesh,
        scratch_types=[
            pltpu.VMEM((2 * _NR, 128), jnp.int32),
            pltpu.VMEM((128, 16), jnp.float32),
            pltpu.VMEM((_PKT, 128), jnp.float32),
            pltpu.VMEM((_NSL, 16), jnp.float32),
            pltpu.VMEM_SHARED((_NP, 16), jnp.float32),
            pltpu.VMEM_SHARED((_NP, 16), jnp.float32),
            pltpu.SemaphoreType.DMA((8,)),
        ],
        compiler_params=_sc_params,
    )
    def deg(srcp, dstp, out, idx_v, ones_v, buf, rows, dsrc, ddst, sems):
        c = lax.axis_index("c")
        s = lax.axis_index("s")
        wid = c * _NS + s
        row0 = wid * _NR
        nd0 = s * _NSL
        pltpu.sync_copy(srcp.at[pl.ds(row0, _NR)], idx_v.at[pl.ds(0, _NR)])
        pltpu.sync_copy(dstp.at[pl.ds(row0, _NR)], idx_v.at[pl.ds(_NR, _NR)])
        one = jnp.full((16,), 1.0, jnp.float32)
        zero = jnp.zeros((16,), jnp.float32)
        for k in range(128):
            ones_v[k, :] = one
        for k in range(_NSL):
            rows[k, :] = zero
        pltpu.sync_copy(rows, dsrc.at[pl.ds(nd0, _NSL)])
        pltpu.sync_copy(rows, ddst.at[pl.ds(nd0, _NSL)])
        plsc.subcore_barrier()
        descs = [None] * 8
        for j in range(2 * _NR):
            k = j % 8
            if descs[k] is not None:
                descs[k].wait()
            tgt = dsrc if j < _NR else ddst
            descs[k] = pltpu.async_copy(ones_v, tgt.at[idx_v.at[j]],
                                        sems.at[k], add=True)
        for k in range(8):
            descs[k].wait()
        plsc.subcore_barrier()
        for which, acc in ((0, dsrc), (1, ddst)):
            pltpu.sync_copy(acc.at[pl.ds(nd0, _NSL)], rows)
            for i in range(_PKT):
                for j in range(8):
                    buf[i, pl.ds(16 * j, 16)] = rows[8 * i + j, :]
            pltpu.sync_copy(buf, out.at[c, which, pl.ds(s * _PKT, _PKT)])

    return deg


def _make_prop(w):
    @functools.partial(
        pl.kernel,
        out_type=jax.ShapeDtypeStruct((_NC, _PK, 128), jnp.float32),
        mesh=_mesh,
        scratch_types=[
            pltpu.VMEM((2 * _NR, 128), jnp.int32),
            pltpu.VMEM((_B, 128, w), jnp.float32),
            pltpu.VMEM((_PKT, 128), jnp.float32),
            pltpu.VMEM((_NSL, w), jnp.float32),
            pltpu.VMEM_SHARED((_NP, w), jnp.float32),
            pltpu.VMEM_SHARED((_NP, w), jnp.float32),
            pltpu.SemaphoreType.DMA((2 * _B,)),
        ],
        compiler_params=_sc_params,
    )
    def prop(table_hbm, srcp, dstp, out, idx_v, rb, buf, rows, table, acc, sems):
        c = lax.axis_index("c")
        s = lax.axis_index("s")
        wid = c * _NS + s
        row0 = wid * _NR
        nd0 = s * _NSL
        pltpu.sync_copy(table_hbm.at[pl.ds(s * _PKT, _PKT)], buf)
        pltpu.sync_copy(srcp.at[pl.ds(row0, _NR)], idx_v.at[pl.ds(0, _NR)])
        pltpu.sync_copy(dstp.at[pl.ds(row0, _NR)], idx_v.at[pl.ds(_NR, _NR)])
        for i in range(_PKT):
            for j in range(8):
                rows[8 * i + j, :] = buf[i, pl.ds(16 * j, 16)]
        pltpu.sync_copy(rows, table.at[pl.ds(nd0, _NSL)])
        zero = jnp.zeros((w,), jnp.float32)
        for k in range(_NSL):
            rows[k, :] = zero
        pltpu.sync_copy(rows, acc.at[pl.ds(nd0, _NSL)])
        plsc.subcore_barrier()
        gd = [None] * _B
        sd = [None] * _B
        for t in range(_NR + _LAG):
            if t >= _LAG:
                j = t - _LAG
                slot = j % _B
                gd[slot].wait()
                sd[slot] = pltpu.async_copy(rb.at[slot], acc.at[idx_v.at[_NR + j]],
                                            sems.at[_B + slot], add=True)
            if t < _NR:
                slot = t % _B
                if sd[slot] is not None:
                    sd[slot].wait()
                    sd[slot] = None
                gd[slot] = pltpu.async_copy(table.at[idx_v.at[t]], rb.at[slot],
                                            sems.at[slot])
        for slot in range(_B):
            if sd[slot] is not None:
                sd[slot].wait()
        plsc.subcore_barrier()
        pltpu.sync_copy(acc.at[pl.ds(nd0, _NSL)], rows)
        for i in range(_PKT):
            for j in range(8):
                buf[i, pl.ds(16 * j, 16)] = rows[8 * i + j, :]
        pltpu.sync_copy(buf, out.at[c, pl.ds(s * _PKT, _PKT)])

    return prop


_DEG = _make_deg()
_P16 = _make_prop(16)


def _norm(deg):
    return jnp.where(deg > 0, lax.rsqrt(jnp.maximum(deg, 1e-12)), 0.0)


def _tca_body(feat_ref, w1bd_ref, degs_ref, u0_ref, ns_ref, nd_ref):
    ns = _norm(degs_ref[0, 0] + degs_ref[1, 0])
    nd = _norm(degs_ref[0, 1] + degs_ref[1, 1])
    y = jnp.dot(feat_ref[...], w1bd_ref[...], preferred_element_type=jnp.float32)
    u0_ref[...] = y * ns
    ns_ref[...] = ns
    nd_ref[...] = nd


def _tct2_body(parts_ref, ns_ref, nd_ref, b_ref, h_ref, u_ref):
    agg = parts_ref[0] + parts_ref[1]
    h = jnp.maximum(agg * nd_ref[...] + b_ref[...], 0.0)
    h_ref[...] = h
    u_ref[...] = h * ns_ref[...]


def _tct3_body(parts_ref, wbd_ref, ns_ref, nd_ref, b_ref, h_ref, u_ref):
    agg = jnp.dot(parts_ref[0] + parts_ref[1], wbd_ref[...],
                  preferred_element_type=jnp.float32)
    h = jnp.maximum(agg * nd_ref[...] + b_ref[...], 0.0)
    h_ref[...] = h
    u_ref[...] = h * ns_ref[...]


def _tct4_body(parts_ref, wbd_ref, nd_ref, b_ref, h_ref):
    agg = jnp.dot(parts_ref[0] + parts_ref[1], wbd_ref[...],
                  preferred_element_type=jnp.float32)
    h_ref[...] = jnp.maximum(agg * nd_ref[...] + b_ref[...], 0.0)


def _tcf_body(pa_ref, pb_ref, pc_ref, w1_ref, w2_ref, w3_ref, bm_ref, out_ref):
    out_ref[...] = (
        jnp.dot(pa_ref[0] + pa_ref[1], w1_ref[...],
                preferred_element_type=jnp.float32)
        + jnp.dot(pb_ref[0] + pb_ref[1], w2_ref[...],
                  preferred_element_type=jnp.float32)
        + jnp.dot(pc_ref[0] + pc_ref[1], w3_ref[...],
                  preferred_element_type=jnp.float32)
        + bm_ref[...])


def _sds(shape):
    return jax.ShapeDtypeStruct(shape, jnp.float32)


def _bd8(w):
    return jax.scipy.linalg.block_diag(*([w] * 8))


def kernel(feat, edge_index, W1, b1, W2, b2, W3, b3, Wm, bm):
    f32 = jnp.float32
    pad_ids = _N + (jnp.arange(_EP - _E, dtype=jnp.int32) % (_NP - _N))
    srcp = jnp.concatenate([edge_index[0], pad_ids]).reshape(_ROWS, 128)
    dstp = jnp.concatenate([edge_index[1], pad_ids]).reshape(_ROWS, 128)
    din = feat.shape[1]
    featp = jnp.concatenate(
        [feat.astype(f32), jnp.zeros((_NP - _N, din), f32)], axis=0)
    feat8 = featp.reshape(_PK, 8 * din)
    w1bd = _bd8(W1)
    w2bd = _bd8(W2)
    w3bd = _bd8(W3)
    wm1bd = _bd8(Wm[0:16])
    wm2bd = _bd8(Wm[16:32])
    wm3bd = _bd8(Wm[32:48])
    b1t = jnp.tile(b1, 8).reshape(1, 128)
    b2t = jnp.tile(b2, 8).reshape(1, 128)
    b3t = jnp.tile(b3, 8).reshape(1, 128)
    bmt = jnp.tile(bm, 8).reshape(1, 1024)

    degs = _DEG(srcp, dstp)
    u0, ns, nd = pl.pallas_call(
        _tca_body,
        out_shape=(_sds((_PK, 128)), _sds((_PK, 128)), _sds((_PK, 128))),
    )(feat8, w1bd, degs)

    p1 = _P16(u0, srcp, dstp)
    h1, u1 = pl.pallas_call(
        _tct2_body, out_shape=(_sds((_PK, 128)), _sds((_PK, 128))),
    )(p1, ns, nd, b1t)

    p2 = _P16(u1, srcp, dstp)
    h2, u2 = pl.pallas_call(
        _tct3_body, out_shape=(_sds((_PK, 128)), _sds((_PK, 128))),
    )(p2, w2bd, ns, nd, b2t)

    p3 = _P16(u2, srcp, dstp)
    h3 = pl.pallas_call(
        _tct4_body, out_shape=_sds((_PK, 128)),
    )(p3, w3bd, nd, b3t)

    p4a = _P16(h1, srcp, dstp)
    p4b = _P16(h2, srcp, dstp)
    p4c = _P16(h3, srcp, dstp)
    outp = pl.pallas_call(
        _tcf_body, out_shape=_sds((_PK, 1024)),
    )(p4a, p4b, p4c, wm1bd, wm2bd, wm3bd, bmt)
    return outp.reshape(_NP, 128)[:_N]


# speedup vs baseline: 26.4270x; 1.0164x over previous
import functools

import jax
import jax.numpy as jnp
from jax import lax
from jax.experimental import pallas as pl
from jax.experimental.pallas import tpu as pltpu
from jax.experimental.pallas import tpu_sc as plsc

_N = 10000
_NP = 10240
_PK = _NP * 16 // 128
_E = 320000
_EP = 327680
_ROWS = _EP // 128
_NC = 2
_NS = 16
_NW = _NC * _NS
_NR = _ROWS // _NW
_NSL = _NP // _NS
_PKT = _PK // _NS
_B = 4
_LAG = 2

_mesh = plsc.VectorSubcoreMesh(core_axis_name="c", subcore_axis_name="s")
_sc_params = pltpu.CompilerParams(use_tc_tiling_on_sc=False)


def _make_deg():
    @functools.partial(
        pl.kernel,
        out_type=jax.ShapeDtypeStruct((_NC * 2 * _NP,), jnp.float32),
        mesh=_mesh,
        scratch_types=[
            pltpu.VMEM((2 * _NR, 128), jnp.int32),
            pltpu.VMEM((128,), jnp.float32),
            pltpu.VMEM_SHARED((_NP,), jnp.float32),
            pltpu.VMEM_SHARED((_NP,), jnp.float32),
            pltpu.SemaphoreType.DMA((8,)),
        ],
        compiler_params=_sc_params,
    )
    def deg(srcp, dstp, zdeg, out, idx_v, ones_v, dsrc, ddst, sems):
        c = lax.axis_index("c")
        s = lax.axis_index("s")
        wid = c * _NS + s
        row0 = wid * _NR
        nd0 = s * _NSL
        pltpu.sync_copy(zdeg.at[pl.ds(nd0, _NSL)], dsrc.at[pl.ds(nd0, _NSL)])
        pltpu.sync_copy(zdeg.at[pl.ds(nd0, _NSL)], ddst.at[pl.ds(nd0, _NSL)])
        pltpu.sync_copy(srcp.at[pl.ds(row0, _NR)], idx_v.at[pl.ds(0, _NR)])
        pltpu.sync_copy(dstp.at[pl.ds(row0, _NR)], idx_v.at[pl.ds(_NR, _NR)])
        for i in range(8):
            ones_v[pl.ds(i * 16, 16)] = jnp.full((16,), 1.0, jnp.float32)
        plsc.subcore_barrier()
        descs = [None] * 8
        for j in range(2 * _NR):
            k = j % 8
            if descs[k] is not None:
                descs[k].wait()
            tgt = dsrc if j < _NR else ddst
            descs[k] = pltpu.async_copy(ones_v, tgt.at[idx_v.at[j]],
                                        sems.at[k], add=True)
        for k in range(8):
            descs[k].wait()
        plsc.subcore_barrier()
        pltpu.sync_copy(dsrc.at[pl.ds(nd0, _NSL)],
                        out.at[pl.ds(c * 2 * _NP + nd0, _NSL)])
        pltpu.sync_copy(ddst.at[pl.ds(nd0, _NSL)],
                        out.at[pl.ds(c * 2 * _NP + _NP + nd0, _NSL)])

    return deg


def _make_prop(w):
    @functools.partial(
        pl.kernel,
        out_type=jax.ShapeDtypeStruct((_NC, _PK, 128), jnp.float32),
        mesh=_mesh,
        scratch_types=[
            pltpu.VMEM((2 * _NR, 128), jnp.int32),
            pltpu.VMEM((_B, 128, w), jnp.float32),
            pltpu.VMEM((_PKT, 128), jnp.float32),
            pltpu.VMEM((_NSL, w), jnp.float32),
            pltpu.VMEM_SHARED((_NP, w), jnp.float32),
            pltpu.VMEM_SHARED((_NP, w), jnp.float32),
            pltpu.SemaphoreType.DMA((2 * _B,)),
        ],
        compiler_params=_sc_params,
    )
    def prop(table_hbm, srcp, dstp, out, idx_v, rb, buf, rows, table, acc, sems):
        c = lax.axis_index("c")
        s = lax.axis_index("s")
        wid = c * _NS + s
        row0 = wid * _NR
        nd0 = s * _NSL
        pltpu.sync_copy(table_hbm.at[pl.ds(s * _PKT, _PKT)], buf)
        pltpu.sync_copy(srcp.at[pl.ds(row0, _NR)], idx_v.at[pl.ds(0, _NR)])
        pltpu.sync_copy(dstp.at[pl.ds(row0, _NR)], idx_v.at[pl.ds(_NR, _NR)])
        for i in range(_PKT):
            for j in range(8):
                rows[8 * i + j, :] = buf[i, pl.ds(16 * j, 16)]
        pltpu.sync_copy(rows, table.at[pl.ds(nd0, _NSL)])
        zero = jnp.zeros((w,), jnp.float32)
        for k in range(_NSL):
            rows[k, :] = zero
        pltpu.sync_copy(rows, acc.at[pl.ds(nd0, _NSL)])
        plsc.subcore_barrier()
        gd = [None] * _B
        sd = [None] * _B
        for t in range(_NR + _LAG):
            if t >= _LAG:
                j = t - _LAG
                slot = j % _B
                gd[slot].wait()
                sd[slot] = pltpu.async_copy(rb.at[slot], acc.at[idx_v.at[_NR + j]],
                                            sems.at[_B + slot], add=True)
            if t < _NR:
                slot = t % _B
                if sd[slot] is not None:
                    sd[slot].wait()
                    sd[slot] = None
                gd[slot] = pltpu.async_copy(table.at[idx_v.at[t]], rb.at[slot],
                                            sems.at[slot])
        for slot in range(_B):
            if sd[slot] is not None:
                sd[slot].wait()
        plsc.subcore_barrier()
        pltpu.sync_copy(acc.at[pl.ds(nd0, _NSL)], rows)
        for i in range(_PKT):
            for j in range(8):
                buf[i, pl.ds(16 * j, 16)] = rows[8 * i + j, :]
        pltpu.sync_copy(buf, out.at[c, pl.ds(s * _PKT, _PKT)])

    return prop


_DEG = _make_deg()
_P16 = _make_prop(16)


def _norm(deg):
    return jnp.where(deg > 0, lax.rsqrt(jnp.maximum(deg, 1e-12)), 0.0)


def _tca_body(feat_ref, w1bd_ref, degs_ref, rep_ref, u0_ref, ns_ref, nd_ref):
    ns = _norm(jnp.dot(degs_ref[0, 0] + degs_ref[1, 0], rep_ref[...],
                       preferred_element_type=jnp.float32))
    nd = _norm(jnp.dot(degs_ref[0, 1] + degs_ref[1, 1], rep_ref[...],
                       preferred_element_type=jnp.float32))
    y = jnp.dot(feat_ref[...], w1bd_ref[...], preferred_element_type=jnp.float32)
    u0_ref[...] = y * ns
    ns_ref[...] = ns
    nd_ref[...] = nd


def _tct2_body(parts_ref, ns_ref, nd_ref, b_ref, h_ref, u_ref):
    agg = parts_ref[0] + parts_ref[1]
    h = jnp.maximum(agg * nd_ref[...] + b_ref[...], 0.0)
    h_ref[...] = h
    u_ref[...] = h * ns_ref[...]


def _tct3_body(parts_ref, wbd_ref, ns_ref, nd_ref, b_ref, h_ref, u_ref):
    agg = jnp.dot(parts_ref[0] + parts_ref[1], wbd_ref[...],
                  preferred_element_type=jnp.float32)
    h = jnp.maximum(agg * nd_ref[...] + b_ref[...], 0.0)
    h_ref[...] = h
    u_ref[...] = h * ns_ref[...]


def _tct4_body(parts_ref, wbd_ref, nd_ref, b_ref, h_ref):
    agg = jnp.dot(parts_ref[0] + parts_ref[1], wbd_ref[...],
                  preferred_element_type=jnp.float32)
    h_ref[...] = jnp.maximum(agg * nd_ref[...] + b_ref[...], 0.0)


def _tcf_body(pa_ref, pb_ref, pc_ref, w1_ref, w2_ref, w3_ref, bm_ref, out_ref):
    out_ref[...] = (
        jnp.dot(pa_ref[0] + pa_ref[1], w1_ref[...],
                preferred_element_type=jnp.float32)
        + jnp.dot(pb_ref[0] + pb_ref[1], w2_ref[...],
                  preferred_element_type=jnp.float32)
        + jnp.dot(pc_ref[0] + pc_ref[1], w3_ref[...],
                  preferred_element_type=jnp.float32)
        + bm_ref[...])


def _sds(shape):
    return jax.ShapeDtypeStruct(shape, jnp.float32)


def _bd8(w):
    return jax.scipy.linalg.block_diag(*([w] * 8))


def kernel(feat, edge_index, W1, b1, W2, b2, W3, b3, Wm, bm):
    f32 = jnp.float32
    pad_ids = _N + jnp.tile(jnp.arange(_NP - _N, dtype=jnp.int32),
                            (_EP - _E) // (_NP - _N))
    ep = jnp.concatenate(
        [edge_index, jnp.broadcast_to(pad_ids, (2, _EP - _E))], axis=1
    ).reshape(2, _ROWS, 128)
    srcp = ep[0]
    dstp = ep[1]
    din = feat.shape[1]
    featp = jnp.concatenate(
        [feat.astype(f32), jnp.zeros((_NP - _N, din), f32)], axis=0)
    feat8 = featp.reshape(_PK, 8 * din)
    w1bd = _bd8(W1)
    w2bd = _bd8(W2)
    w3bd = _bd8(W3)
    wm1bd = _bd8(Wm[0:16])
    wm2bd = _bd8(Wm[16:32])
    wm3bd = _bd8(Wm[32:48])
    b1t = jnp.tile(b1, 8).reshape(1, 128)
    b2t = jnp.tile(b2, 8).reshape(1, 128)
    b3t = jnp.tile(b3, 8).reshape(1, 128)
    bmt = jnp.tile(bm, 8).reshape(1, 1024)
    rep = (jnp.arange(128, dtype=jnp.int32) // 16 ==
           jnp.arange(8, dtype=jnp.int32)[:, None]).astype(f32)
    zdeg = jnp.zeros((_NP,), f32)

    degs = _DEG(srcp, dstp, zdeg).reshape(_NC, 2, _PK, 8)
    u0, ns, nd = pl.pallas_call(
        _tca_body,
        out_shape=(_sds((_PK, 128)), _sds((_PK, 128)), _sds((_PK, 128))),
    )(feat8, w1bd, degs, rep)

    p1 = _P16(u0, srcp, dstp)
    h1, u1 = pl.pallas_call(
        _tct2_body, out_shape=(_sds((_PK, 128)), _sds((_PK, 128))),
    )(p1, ns, nd, b1t)

    p2 = _P16(u1, srcp, dstp)
    h2, u2 = pl.pallas_call(
        _tct3_body, out_shape=(_sds((_PK, 128)), _sds((_PK, 128))),
    )(p2, w2bd, ns, nd, b2t)

    p3 = _P16(u2, srcp, dstp)
    h3 = pl.pallas_call(
        _tct4_body, out_shape=_sds((_PK, 128)),
    )(p3, w3bd, nd, b3t)

    p4a = _P16(h1, srcp, dstp)
    p4b = _P16(h2, srcp, dstp)
    p4c = _P16(h3, srcp, dstp)
    outp = pl.pallas_call(
        _tcf_body, out_shape=_sds((_PK, 1024)),
    )(p4a, p4b, p4c, wm1bd, wm2bd, wm3bd, bmt)
    return outp.reshape(_NP, 128)[:_N]


# speedup vs baseline: 27.3561x; 1.0352x over previous
import functools

import jax
import jax.numpy as jnp
from jax import lax
from jax.experimental import pallas as pl
from jax.experimental.pallas import tpu as pltpu
from jax.experimental.pallas import tpu_sc as plsc

_N = 10000
_NP = 10240
_PK = _NP * 16 // 128
_E = 320000
_EP = 327680
_ROWS = _EP // 128
_NC = 2
_NS = 16
_NW = _NC * _NS
_NR = _ROWS // _NW
_NSL = _NP // _NS
_PKT = _PK // _NS
_B = 4
_LAG = 2

_mesh = plsc.VectorSubcoreMesh(core_axis_name="c", subcore_axis_name="s")
_sc_params = pltpu.CompilerParams(use_tc_tiling_on_sc=False)


def _make_deg():
    @functools.partial(
        pl.kernel,
        out_type=jax.ShapeDtypeStruct((_NC * 2 * _NP,), jnp.float32),
        mesh=_mesh,
        scratch_types=[
            pltpu.VMEM((2 * _NR, 128), jnp.int32),
            pltpu.VMEM((128,), jnp.float32),
            pltpu.VMEM_SHARED((_NP,), jnp.float32),
            pltpu.VMEM_SHARED((_NP,), jnp.float32),
            pltpu.SemaphoreType.DMA((8,)),
        ],
        compiler_params=_sc_params,
    )
    def deg(ep, zdeg, out, idx_v, ones_v, dsrc, ddst, sems):
        c = lax.axis_index("c")
        s = lax.axis_index("s")
        wid = c * _NS + s
        row0 = wid * _NR
        nd0 = s * _NSL
        pltpu.sync_copy(zdeg.at[pl.ds(nd0, _NSL)], dsrc.at[pl.ds(nd0, _NSL)])
        pltpu.sync_copy(zdeg.at[pl.ds(nd0, _NSL)], ddst.at[pl.ds(nd0, _NSL)])
        pltpu.sync_copy(ep.at[0, pl.ds(row0, _NR)], idx_v.at[pl.ds(0, _NR)])
        pltpu.sync_copy(ep.at[1, pl.ds(row0, _NR)], idx_v.at[pl.ds(_NR, _NR)])
        for i in range(8):
            ones_v[pl.ds(i * 16, 16)] = jnp.full((16,), 1.0, jnp.float32)
        plsc.subcore_barrier()
        descs = [None] * 8
        for j in range(2 * _NR):
            k = j % 8
            if descs[k] is not None:
                descs[k].wait()
            tgt = dsrc if j < _NR else ddst
            descs[k] = pltpu.async_copy(ones_v, tgt.at[idx_v.at[j]],
                                        sems.at[k], add=True)
        for k in range(8):
            descs[k].wait()
        plsc.subcore_barrier()
        pltpu.sync_copy(dsrc.at[pl.ds(nd0, _NSL)],
                        out.at[pl.ds(c * 2 * _NP + nd0, _NSL)])
        pltpu.sync_copy(ddst.at[pl.ds(nd0, _NSL)],
                        out.at[pl.ds(c * 2 * _NP + _NP + nd0, _NSL)])

    return deg


def _make_prop(w):
    @functools.partial(
        pl.kernel,
        out_type=jax.ShapeDtypeStruct((_NC, _PK, 128), jnp.float32),
        mesh=_mesh,
        scratch_types=[
            pltpu.VMEM((2 * _NR, 128), jnp.int32),
            pltpu.VMEM((_B, 128, w), jnp.float32),
            pltpu.VMEM((_PKT, 128), jnp.float32),
            pltpu.VMEM((_NSL, w), jnp.float32),
            pltpu.VMEM_SHARED((_NP, w), jnp.float32),
            pltpu.VMEM_SHARED((_NP, w), jnp.float32),
            pltpu.SemaphoreType.DMA((2 * _B,)),
        ],
        compiler_params=_sc_params,
    )
    def prop(table_hbm, ep, out, idx_v, rb, buf, rows, table, acc, sems):
        c = lax.axis_index("c")
        s = lax.axis_index("s")
        wid = c * _NS + s
        row0 = wid * _NR
        nd0 = s * _NSL
        pltpu.sync_copy(table_hbm.at[pl.ds(s * _PKT, _PKT)], buf)
        pltpu.sync_copy(ep.at[0, pl.ds(row0, _NR)], idx_v.at[pl.ds(0, _NR)])
        pltpu.sync_copy(ep.at[1, pl.ds(row0, _NR)], idx_v.at[pl.ds(_NR, _NR)])
        for i in range(_PKT):
            for j in range(8):
                rows[8 * i + j, :] = buf[i, pl.ds(16 * j, 16)]
        pltpu.sync_copy(rows, table.at[pl.ds(nd0, _NSL)])
        zero = jnp.zeros((w,), jnp.float32)
        for k in range(_NSL):
            rows[k, :] = zero
        pltpu.sync_copy(rows, acc.at[pl.ds(nd0, _NSL)])
        plsc.subcore_barrier()
        gd = [None] * _B
        sd = [None] * _B
        for t in range(_NR + _LAG):
            if t >= _LAG:
                j = t - _LAG
                slot = j % _B
                gd[slot].wait()
                sd[slot] = pltpu.async_copy(rb.at[slot], acc.at[idx_v.at[_NR + j]],
                                            sems.at[_B + slot], add=True)
            if t < _NR:
                slot = t % _B
                if sd[slot] is not None:
                    sd[slot].wait()
                    sd[slot] = None
                gd[slot] = pltpu.async_copy(table.at[idx_v.at[t]], rb.at[slot],
                                            sems.at[slot])
        for slot in range(_B):
            if sd[slot] is not None:
                sd[slot].wait()
        plsc.subcore_barrier()
        pltpu.sync_copy(acc.at[pl.ds(nd0, _NSL)], rows)
        for i in range(_PKT):
            for j in range(8):
                buf[i, pl.ds(16 * j, 16)] = rows[8 * i + j, :]
        pltpu.sync_copy(buf, out.at[c, pl.ds(s * _PKT, _PKT)])

    return prop


_DEG = _make_deg()
_P16 = _make_prop(16)


def _norm(deg):
    return jnp.where(deg > 0, lax.rsqrt(jnp.maximum(deg, 1e-12)), 0.0)


def _tca_body(feat_ref, w1bd_ref, degs_ref, rep_ref, u0_ref, ns_ref, nd_ref):
    ns = _norm(jnp.dot(degs_ref[0, 0] + degs_ref[1, 0], rep_ref[...],
                       preferred_element_type=jnp.float32))
    nd = _norm(jnp.dot(degs_ref[0, 1] + degs_ref[1, 1], rep_ref[...],
                       preferred_element_type=jnp.float32))
    y = jnp.dot(feat_ref[...], w1bd_ref[...], preferred_element_type=jnp.float32)
    u0_ref[...] = y * ns
    ns_ref[...] = ns
    nd_ref[...] = nd


def _tct2_body(parts_ref, ns_ref, nd_ref, b_ref, h_ref, u_ref):
    agg = parts_ref[0] + parts_ref[1]
    h = jnp.maximum(agg * nd_ref[...] + b_ref[...], 0.0)
    h_ref[...] = h
    u_ref[...] = h * ns_ref[...]


def _tct3_body(parts_ref, wbd_ref, ns_ref, nd_ref, b_ref, h_ref, u_ref):
    agg = jnp.dot(parts_ref[0] + parts_ref[1], wbd_ref[...],
                  preferred_element_type=jnp.float32)
    h = jnp.maximum(agg * nd_ref[...] + b_ref[...], 0.0)
    h_ref[...] = h
    u_ref[...] = h * ns_ref[...]


def _tct4_body(parts_ref, wbd_ref, nd_ref, b_ref, h_ref):
    agg = jnp.dot(parts_ref[0] + parts_ref[1], wbd_ref[...],
                  preferred_element_type=jnp.float32)
    h_ref[...] = jnp.maximum(agg * nd_ref[...] + b_ref[...], 0.0)


def _tcf_body(pa_ref, pb_ref, pc_ref, w1_ref, w2_ref, w3_ref, bm_ref, out_ref):
    out_ref[...] = (
        jnp.dot(pa_ref[0] + pa_ref[1], w1_ref[...],
                preferred_element_type=jnp.float32)
        + jnp.dot(pb_ref[0] + pb_ref[1], w2_ref[...],
                  preferred_element_type=jnp.float32)
        + jnp.dot(pc_ref[0] + pc_ref[1], w3_ref[...],
                  preferred_element_type=jnp.float32)
        + bm_ref[...])


def _sds(shape):
    return jax.ShapeDtypeStruct(shape, jnp.float32)


def _bd8(w):
    return jax.scipy.linalg.block_diag(*([w] * 8))


def kernel(feat, edge_index, W1, b1, W2, b2, W3, b3, Wm, bm):
    f32 = jnp.float32
    pad_ids = _N + jnp.tile(jnp.arange(_NP - _N, dtype=jnp.int32),
                            (_EP - _E) // (_NP - _N))
    ep = jnp.concatenate(
        [edge_index, jnp.broadcast_to(pad_ids, (2, _EP - _E))], axis=1
    ).reshape(2, _ROWS, 128)
    din = feat.shape[1]
    featp = jnp.concatenate(
        [feat.astype(f32), jnp.zeros((_NP - _N, din), f32)], axis=0)
    feat8 = featp.reshape(_PK, 8 * din)
    w1bd = _bd8(W1)
    w2bd = _bd8(W2)
    w3bd = _bd8(W3)
    wm1bd = _bd8(Wm[0:16])
    wm2bd = _bd8(Wm[16:32])
    wm3bd = _bd8(Wm[32:48])
    b1t = jnp.tile(b1, 8).reshape(1, 128)
    b2t = jnp.tile(b2, 8).reshape(1, 128)
    b3t = jnp.tile(b3, 8).reshape(1, 128)
    bmt = jnp.tile(bm, 8).reshape(1, 1024)
    rep = (jnp.arange(128, dtype=jnp.int32) // 16 ==
           jnp.arange(8, dtype=jnp.int32)[:, None]).astype(f32)
    zdeg = jnp.zeros((_NP,), f32)

    degs = _DEG(ep, zdeg).reshape(_NC, 2, _PK, 8)
    u0, ns, nd = pl.pallas_call(
        _tca_body,
        out_shape=(_sds((_PK, 128)), _sds((_PK, 128)), _sds((_PK, 128))),
    )(feat8, w1bd, degs, rep)

    p1 = _P16(u0, ep)
    h1, u1 = pl.pallas_call(
        _tct2_body, out_shape=(_sds((_PK, 128)), _sds((_PK, 128))),
    )(p1, ns, nd, b1t)

    p2 = _P16(u1, ep)
    h2, u2 = pl.pallas_call(
        _tct3_body, out_shape=(_sds((_PK, 128)), _sds((_PK, 128))),
    )(p2, w2bd, ns, nd, b2t)

    p3 = _P16(u2, ep)
    h3 = pl.pallas_call(
        _tct4_body, out_shape=_sds((_PK, 128)),
    )(p3, w3bd, nd, b3t)

    p4a = _P16(h1, ep)
    p4b = _P16(h2, ep)
    p4c = _P16(h3, ep)
    outp = pl.pallas_call(
        _tcf_body, out_shape=_sds((_PK, 1024)),
    )(p4a, p4b, p4c, wm1bd, wm2bd, wm3bd, bmt)
    return outp.reshape(_NP, 128)[:_N]


# speedup vs baseline: 28.3045x; 1.0347x over previous
import functools

import jax
import jax.numpy as jnp
from jax import lax
from jax.experimental import pallas as pl
from jax.experimental.pallas import tpu as pltpu
from jax.experimental.pallas import tpu_sc as plsc

_N = 10000
_NP = 10240
_PK = _NP * 16 // 128
_E = 320000
_EP = 327680
_ROWS = _EP // 128
_NC = 2
_NS = 16
_NW = _NC * _NS
_NR = _ROWS // _NW
_NSL = _NP // _NS
_PKT = _PK // _NS
_B = 6
_LAG = 3

_mesh = plsc.VectorSubcoreMesh(core_axis_name="c", subcore_axis_name="s")
_sc_params = pltpu.CompilerParams(use_tc_tiling_on_sc=False)


def _make_deg():
    @functools.partial(
        pl.kernel,
        out_type=jax.ShapeDtypeStruct((_NC * 2 * _NP,), jnp.float32),
        mesh=_mesh,
        scratch_types=[
            pltpu.VMEM((2 * _NR, 128), jnp.int32),
            pltpu.VMEM((128,), jnp.float32),
            pltpu.VMEM_SHARED((_NP,), jnp.float32),
            pltpu.VMEM_SHARED((_NP,), jnp.float32),
            pltpu.SemaphoreType.DMA((8,)),
        ],
        compiler_params=_sc_params,
    )
    def deg(ep, zdeg, out, idx_v, ones_v, dsrc, ddst, sems):
        c = lax.axis_index("c")
        s = lax.axis_index("s")
        wid = c * _NS + s
        row0 = wid * _NR
        nd0 = s * _NSL
        pltpu.sync_copy(zdeg.at[pl.ds(nd0, _NSL)], dsrc.at[pl.ds(nd0, _NSL)])
        pltpu.sync_copy(zdeg.at[pl.ds(nd0, _NSL)], ddst.at[pl.ds(nd0, _NSL)])
        pltpu.sync_copy(ep.at[0, pl.ds(row0, _NR)], idx_v.at[pl.ds(0, _NR)])
        pltpu.sync_copy(ep.at[1, pl.ds(row0, _NR)], idx_v.at[pl.ds(_NR, _NR)])
        for i in range(8):
            ones_v[pl.ds(i * 16, 16)] = jnp.full((16,), 1.0, jnp.float32)
        plsc.subcore_barrier()
        descs = [None] * 8
        for j in range(2 * _NR):
            k = j % 8
            if descs[k] is not None:
                descs[k].wait()
            tgt = dsrc if j < _NR else ddst
            descs[k] = pltpu.async_copy(ones_v, tgt.at[idx_v.at[j]],
                                        sems.at[k], add=True)
        for k in range(8):
            descs[k].wait()
        plsc.subcore_barrier()
        pltpu.sync_copy(dsrc.at[pl.ds(nd0, _NSL)],
                        out.at[pl.ds(c * 2 * _NP + nd0, _NSL)])
        pltpu.sync_copy(ddst.at[pl.ds(nd0, _NSL)],
                        out.at[pl.ds(c * 2 * _NP + _NP + nd0, _NSL)])

    return deg


def _make_prop(w):
    @functools.partial(
        pl.kernel,
        out_type=jax.ShapeDtypeStruct((_NC, _PK, 128), jnp.float32),
        mesh=_mesh,
        scratch_types=[
            pltpu.VMEM((2 * _NR, 128), jnp.int32),
            pltpu.VMEM((_B, 128, w), jnp.float32),
            pltpu.VMEM((_PKT, 128), jnp.float32),
            pltpu.VMEM((_NSL, w), jnp.float32),
            pltpu.VMEM_SHARED((_NP, w), jnp.float32),
            pltpu.VMEM_SHARED((_NP, w), jnp.float32),
            pltpu.SemaphoreType.DMA((2 * _B,)),
        ],
        compiler_params=_sc_params,
    )
    def prop(table_hbm, ep, out, idx_v, rb, buf, rows, table, acc, sems):
        c = lax.axis_index("c")
        s = lax.axis_index("s")
        wid = c * _NS + s
        row0 = wid * _NR
        nd0 = s * _NSL
        pltpu.sync_copy(table_hbm.at[pl.ds(s * _PKT, _PKT)], buf)
        pltpu.sync_copy(ep.at[0, pl.ds(row0, _NR)], idx_v.at[pl.ds(0, _NR)])
        pltpu.sync_copy(ep.at[1, pl.ds(row0, _NR)], idx_v.at[pl.ds(_NR, _NR)])
        for i in range(_PKT):
            for j in range(8):
                rows[8 * i + j, :] = buf[i, pl.ds(16 * j, 16)]
        pltpu.sync_copy(rows, table.at[pl.ds(nd0, _NSL)])
        zero = jnp.zeros((w,), jnp.float32)
        for k in range(_NSL):
            rows[k, :] = zero
        pltpu.sync_copy(rows, acc.at[pl.ds(nd0, _NSL)])
        plsc.subcore_barrier()
        gd = [None] * _B
        sd = [None] * _B
        for t in range(_NR + _LAG):
            if t >= _LAG:
                j = t - _LAG
                slot = j % _B
                gd[slot].wait()
                sd[slot] = pltpu.async_copy(rb.at[slot], acc.at[idx_v.at[_NR + j]],
                                            sems.at[_B + slot], add=True)
            if t < _NR:
                slot = t % _B
                if sd[slot] is not None:
                    sd[slot].wait()
                    sd[slot] = None
                gd[slot] = pltpu.async_copy(table.at[idx_v.at[t]], rb.at[slot],
                                            sems.at[slot])
        for slot in range(_B):
            if sd[slot] is not None:
                sd[slot].wait()
        plsc.subcore_barrier()
        pltpu.sync_copy(acc.at[pl.ds(nd0, _NSL)], rows)
        for i in range(_PKT):
            for j in range(8):
                buf[i, pl.ds(16 * j, 16)] = rows[8 * i + j, :]
        pltpu.sync_copy(buf, out.at[c, pl.ds(s * _PKT, _PKT)])

    return prop


_DEG = _make_deg()
_P16 = _make_prop(16)


def _norm(deg):
    return jnp.where(deg > 0, lax.rsqrt(jnp.maximum(deg, 1e-12)), 0.0)


def _tca_body(feat_ref, w1bd_ref, degs_ref, rep_ref, u0_ref, ns_ref, nd_ref):
    ns = _norm(jnp.dot(degs_ref[0, 0] + degs_ref[1, 0], rep_ref[...],
                       preferred_element_type=jnp.float32))
    nd = _norm(jnp.dot(degs_ref[0, 1] + degs_ref[1, 1], rep_ref[...],
                       preferred_element_type=jnp.float32))
    y = jnp.dot(feat_ref[...], w1bd_ref[...], preferred_element_type=jnp.float32)
    u0_ref[...] = y * ns
    ns_ref[...] = ns
    nd_ref[...] = nd


def _tct2_body(parts_ref, ns_ref, nd_ref, b_ref, h_ref, u_ref):
    agg = parts_ref[0] + parts_ref[1]
    h = jnp.maximum(agg * nd_ref[...] + b_ref[...], 0.0)
    h_ref[...] = h
    u_ref[...] = h * ns_ref[...]


def _tct3_body(parts_ref, wbd_ref, ns_ref, nd_ref, b_ref, h_ref, u_ref):
    agg = jnp.dot(parts_ref[0] + parts_ref[1], wbd_ref[...],
                  preferred_element_type=jnp.float32)
    h = jnp.maximum(agg * nd_ref[...] + b_ref[...], 0.0)
    h_ref[...] = h
    u_ref[...] = h * ns_ref[...]


def _tct4_body(parts_ref, wbd_ref, nd_ref, b_ref, h_ref):
    agg = jnp.dot(parts_ref[0] + parts_ref[1], wbd_ref[...],
                  preferred_element_type=jnp.float32)
    h_ref[...] = jnp.maximum(agg * nd_ref[...] + b_ref[...], 0.0)


def _tcf_body(pa_ref, pb_ref, pc_ref, w1_ref, w2_ref, w3_ref, bm_ref, out_ref):
    out_ref[...] = (
        jnp.dot(pa_ref[0] + pa_ref[1], w1_ref[...],
                preferred_element_type=jnp.float32)
        + jnp.dot(pb_ref[0] + pb_ref[1], w2_ref[...],
                  preferred_element_type=jnp.float32)
        + jnp.dot(pc_ref[0] + pc_ref[1], w3_ref[...],
                  preferred_element_type=jnp.float32)
        + bm_ref[...])


def _sds(shape):
    return jax.ShapeDtypeStruct(shape, jnp.float32)


def _bd8(w):
    return jax.scipy.linalg.block_diag(*([w] * 8))


def kernel(feat, edge_index, W1, b1, W2, b2, W3, b3, Wm, bm):
    f32 = jnp.float32
    pad_ids = _N + jnp.tile(jnp.arange(_NP - _N, dtype=jnp.int32),
                            (_EP - _E) // (_NP - _N))
    ep = jnp.concatenate(
        [edge_index, jnp.broadcast_to(pad_ids, (2, _EP - _E))], axis=1
    ).reshape(2, _ROWS, 128)
    din = feat.shape[1]
    featp = jnp.concatenate(
        [feat.astype(f32), jnp.zeros((_NP - _N, din), f32)], axis=0)
    feat8 = featp.reshape(_PK, 8 * din)
    w1bd = _bd8(W1)
    w2bd = _bd8(W2)
    w3bd = _bd8(W3)
    wm1bd = _bd8(Wm[0:16])
    wm2bd = _bd8(Wm[16:32])
    wm3bd = _bd8(Wm[32:48])
    b1t = jnp.tile(b1, 8).reshape(1, 128)
    b2t = jnp.tile(b2, 8).reshape(1, 128)
    b3t = jnp.tile(b3, 8).reshape(1, 128)
    bmt = jnp.tile(bm, 8).reshape(1, 1024)
    rep = (jnp.arange(128, dtype=jnp.int32) // 16 ==
           jnp.arange(8, dtype=jnp.int32)[:, None]).astype(f32)
    zdeg = jnp.zeros((_NP,), f32)

    degs = _DEG(ep, zdeg).reshape(_NC, 2, _PK, 8)
    u0, ns, nd = pl.pallas_call(
        _tca_body,
        out_shape=(_sds((_PK, 128)), _sds((_PK, 128)), _sds((_PK, 128))),
    )(feat8, w1bd, degs, rep)

    p1 = _P16(u0, ep)
    h1, u1 = pl.pallas_call(
        _tct2_body, out_shape=(_sds((_PK, 128)), _sds((_PK, 128))),
    )(p1, ns, nd, b1t)

    p2 = _P16(u1, ep)
    h2, u2 = pl.pallas_call(
        _tct3_body, out_shape=(_sds((_PK, 128)), _sds((_PK, 128))),
    )(p2, w2bd, ns, nd, b2t)

    p3 = _P16(u2, ep)
    h3 = pl.pallas_call(
        _tct4_body, out_shape=_sds((_PK, 128)),
    )(p3, w3bd, nd, b3t)

    p4a = _P16(h1, ep)
    p4b = _P16(h2, ep)
    p4c = _P16(h3, ep)
    outp = pl.pallas_call(
        _tcf_body, out_shape=_sds((_PK, 1024)),
    )(p4a, p4b, p4c, wm1bd, wm2bd, wm3bd, bmt)
    return outp.reshape(_NP, 128)[:_N]
